# baseline (speedup 1.0000x reference)
import functools

import jax
import jax.numpy as jnp
from jax import lax
from jax.experimental import pallas as pl
from jax.experimental.pallas import tpu as pltpu
from jax.experimental.pallas import tpu_sc as plsc

FDIM = 64
GRID = 513
WP = GRID + 2
N_POINTS = 1000000
C = 128
NG = C // 16
NUM_WORKERS = 32
NCHUNK = -(-N_POINTS // C)
TPT = -(-NCHUNK // NUM_WORKERS)


def _sc_body(xs_hbm, ys_hbm, table_hbm, out_hbm,
             xs_s, ys_s, idx_s, w_s, rows_s, out_s,
             sem_c0, sem_c1, sem_g0, sem_g1, sem_o0, sem_o1):
    wid = lax.axis_index("s") * 2 + lax.axis_index("c")
    sem_c = [sem_c0, sem_c1]
    sem_g = [sem_g0, sem_g1]
    sem_o = [sem_o0, sem_o1]

    def chunk_base(t):
        return jnp.minimum((t * NUM_WORKERS + wid) * C, N_POINTS - C)

    def active(t):
        return (t * NUM_WORKERS + wid) < NCHUNK

    def coords_issue(t, b):
        @pl.when(active(t))
        def _():
            base = chunk_base(t)
            pltpu.async_copy(xs_hbm.at[pl.ds(base, C)], xs_s.at[b], sem_c[b])
            pltpu.async_copy(ys_hbm.at[pl.ds(base, C)], ys_s.at[b], sem_c[b])

    def prefetch(t, b):
        @pl.when(active(t))
        def _():
            pltpu.make_async_copy(xs_hbm.at[pl.ds(0, C)], xs_s.at[b],
                                  sem_c[b]).wait()
            pltpu.make_async_copy(ys_hbm.at[pl.ds(0, C)], ys_s.at[b],
                                  sem_c[b]).wait()
            for gi in range(NG):
                sl = pl.ds(gi * 16, 16)
                gx = xs_s[b, sl]
                gy = ys_s[b, sl]
                izx = ((gx + 1.0) * 513.0 - 1.0) * 0.5 + 1.0
                izy = ((gy + 1.0) * 513.0 - 1.0) * 0.5 + 1.0
                x0 = izx.astype(jnp.int32)
                y0 = izy.astype(jnp.int32)
                wx1 = izx - x0.astype(jnp.float32)
                wy1 = izy - y0.astype(jnp.float32)
                wx0 = 1.0 - wx1
                wy0 = 1.0 - wy1
                x0 = jnp.clip(x0, 0, WP - 2)
                y0 = jnp.clip(y0, 0, WP - 2)
                i00 = y0 * WP + x0
                idx_s[b, 0, sl] = i00
                idx_s[b, 1, sl] = i00 + WP
                w_s[b, 0, sl] = wx0 * wy0
                w_s[b, 1, sl] = wx1 * wy0
                w_s[b, 2, sl] = wx0 * wy1
                w_s[b, 3, sl] = wx1 * wy1
            for c in range(2):
                pltpu.async_copy(table_hbm.at[idx_s.at[b, c]],
                                 rows_s.at[b, c], sem_g[b])

    def blend_store(t, b):
        @pl.when(active(t))
        def _():
            for c in range(2):
                pltpu.make_async_copy(table_hbm.at[idx_s.at[b, c]],
                                      rows_s.at[b, c], sem_g[b]).wait()

            @pl.when(t >= 2)
            def _():
                pltpu.make_async_copy(out_s.at[b],
                                      out_hbm.at[pl.ds(0, C)], sem_o[b]).wait()

            def g_body(gi, _):
                p0 = gi * 16
                w0v = w_s[b, 0, pl.ds(p0, 16)]
                w1v = w_s[b, 1, pl.ds(p0, 16)]
                w2v = w_s[b, 2, pl.ds(p0, 16)]
                w3v = w_s[b, 3, pl.ds(p0, 16)]
                for lane in range(16):
                    p = p0 + lane
                    w0 = w0v[lane]
                    w1 = w1v[lane]
                    w2 = w2v[lane]
                    w3 = w3v[lane]
                    for jc in range(FDIM // 16):
                        sl = pl.ds(jc * 16, 16)
                        sl1 = pl.ds(FDIM + jc * 16, 16)
                        v0 = rows_s[b, 0, p, sl]
                        v1 = rows_s[b, 0, p, sl1]
                        v2 = rows_s[b, 1, p, sl]
                        v3 = rows_s[b, 1, p, sl1]
                        out_s[b, p, sl] = (v0 * w0 + v1 * w1
                                           + v2 * w2 + v3 * w3)
                return 0

            lax.fori_loop(0, NG, g_body, 0)
            pltpu.async_copy(out_s.at[b], out_hbm.at[pl.ds(chunk_base(t), C)],
                             sem_o[b])

    coords_issue(0, 0)
    coords_issue(1, 1)
    prefetch(0, 0)
    coords_issue(2, 0)

    def pipe_iter(k, carry):
        t0 = 2 * k
        t1 = 2 * k + 1
        prefetch(t1, 1)
        coords_issue(t1 + 2, 1)
        blend_store(t0, 0)
        prefetch(t0 + 2, 0)
        coords_issue(t0 + 4, 0)
        blend_store(t1, 1)
        return carry

    lax.fori_loop(0, (TPT + 1) // 2, pipe_iter, 0)

    for b in range(2):
        pltpu.make_async_copy(out_s.at[b], out_hbm.at[pl.ds(0, C)],
                              sem_o[b]).wait()


@jax.jit
def _impl(xs, ys, table):
    mesh = plsc.VectorSubcoreMesh(core_axis_name="c", subcore_axis_name="s")
    f = functools.partial(
        pl.kernel,
        mesh=mesh,
        out_type=jax.ShapeDtypeStruct((N_POINTS, FDIM), jnp.float32),
        scratch_types=[
            pltpu.VMEM((2, C), jnp.float32),
            pltpu.VMEM((2, C), jnp.float32),
            pltpu.VMEM((2, 2, C), jnp.int32),
            pltpu.VMEM((2, 4, C), jnp.float32),
            pltpu.VMEM((2, 2, C, 2 * FDIM), jnp.float32),
            pltpu.VMEM((2, C, FDIM), jnp.float32),
            pltpu.SemaphoreType.DMA,
            pltpu.SemaphoreType.DMA,
            pltpu.SemaphoreType.DMA,
            pltpu.SemaphoreType.DMA,
            pltpu.SemaphoreType.DMA,
            pltpu.SemaphoreType.DMA,
        ],
    )(_sc_body)
    return f(xs, ys, table)


def kernel(x, fm):
    xs = x[:, 0]
    ys = x[:, 1]
    eye = jnp.eye(FDIM, dtype=jnp.float32)
    t = jnp.einsum('cyx,cd->yxd', fm[0], eye,
                   preferred_element_type=jnp.float32)
    t = jnp.pad(t, ((1, 1), (1, 1), (0, 0)))
    flat = t.reshape(WP * WP, FDIM)
    table = jnp.concatenate([flat[:-1], flat[1:]], axis=1)
    return _impl(xs, ys, table)


# speedup vs baseline: 1.0310x; 1.0310x over previous
import functools

import jax
import jax.numpy as jnp
from jax import lax
from jax.experimental import pallas as pl
from jax.experimental.pallas import tpu as pltpu
from jax.experimental.pallas import tpu_sc as plsc

FDIM = 64
GRID = 513
WP = GRID + 2
N_POINTS = 1000000
C = 128
NG = C // 16
NUM_WORKERS = 32
NCHUNK = -(-N_POINTS // C)
TPT = -(-NCHUNK // NUM_WORKERS)


def _sc_body(xs_hbm, ys_hbm, table_hbm, out_hbm,
             xs_s, ys_s, idx_s, w_s, rows_s, out_s,
             sem_c0, sem_c1, sem_g0, sem_g1, sem_o0, sem_o1):
    wid = lax.axis_index("s") * 2 + lax.axis_index("c")
    sem_c = [sem_c0, sem_c1]
    sem_g = [sem_g0, sem_g1]
    sem_o = [sem_o0, sem_o1]

    def chunk_base(t):
        return jnp.minimum((t * NUM_WORKERS + wid) * C, N_POINTS - C)

    def active(t):
        return (t * NUM_WORKERS + wid) < NCHUNK

    def coords_issue(t, b):
        @pl.when(active(t))
        def _():
            base = chunk_base(t)
            pltpu.async_copy(xs_hbm.at[pl.ds(base, C)], xs_s.at[b], sem_c[b])
            pltpu.async_copy(ys_hbm.at[pl.ds(base, C)], ys_s.at[b], sem_c[b])

    def prefetch(t, b):
        @pl.when(active(t))
        def _():
            pltpu.make_async_copy(xs_hbm.at[pl.ds(0, C)], xs_s.at[b],
                                  sem_c[b]).wait()
            pltpu.make_async_copy(ys_hbm.at[pl.ds(0, C)], ys_s.at[b],
                                  sem_c[b]).wait()
            for gi in range(NG):
                sl = pl.ds(gi * 16, 16)
                gx = xs_s[b, sl]
                gy = ys_s[b, sl]
                izx = ((gx + 1.0) * 513.0 - 1.0) * 0.5 + 1.0
                izy = ((gy + 1.0) * 513.0 - 1.0) * 0.5 + 1.0
                x0 = izx.astype(jnp.int32)
                y0 = izy.astype(jnp.int32)
                wx1 = izx - x0.astype(jnp.float32)
                wy1 = izy - y0.astype(jnp.float32)
                wx0 = 1.0 - wx1
                wy0 = 1.0 - wy1
                x0 = jnp.clip(x0, 0, WP - 2)
                y0 = jnp.clip(y0, 0, WP - 2)
                i00 = y0 * WP + x0
                idx_s[b, 0, sl] = i00
                idx_s[b, 1, sl] = i00 + WP
                w_s[b, 0, sl] = wx0 * wy0
                w_s[b, 1, sl] = wx1 * wy0
                w_s[b, 2, sl] = wx0 * wy1
                w_s[b, 3, sl] = wx1 * wy1
            for c in range(2):
                pltpu.async_copy(table_hbm.at[idx_s.at[b, c]],
                                 rows_s.at[b, c], sem_g[b])

    def blend_store(t, b):
        @pl.when(active(t))
        def _():
            for c in range(2):
                pltpu.make_async_copy(table_hbm.at[idx_s.at[b, c]],
                                      rows_s.at[b, c], sem_g[b]).wait()

            @pl.when(t >= 2)
            def _():
                pltpu.make_async_copy(out_s.at[b],
                                      out_hbm.at[pl.ds(0, C)], sem_o[b]).wait()

            def g_body(gi, _):
                p0 = gi * 16
                w0v = w_s[b, 0, pl.ds(p0, 16)]
                w1v = w_s[b, 1, pl.ds(p0, 16)]
                w2v = w_s[b, 2, pl.ds(p0, 16)]
                w3v = w_s[b, 3, pl.ds(p0, 16)]
                for lane in range(16):
                    p = p0 + lane
                    w0 = w0v[lane]
                    w1 = w1v[lane]
                    w2 = w2v[lane]
                    w3 = w3v[lane]
                    for jc in range(FDIM // 16):
                        sl = pl.ds(jc * 16, 16)
                        sl1 = pl.ds(FDIM + jc * 16, 16)
                        v0 = rows_s[b, 0, p, sl]
                        v1 = rows_s[b, 0, p, sl1]
                        v2 = rows_s[b, 1, p, sl]
                        v3 = rows_s[b, 1, p, sl1]
                        out_s[b, p, sl] = (v0 * w0 + v1 * w1
                                           + v2 * w2 + v3 * w3)
                return 0

            lax.fori_loop(0, NG, g_body, 0)
            pltpu.async_copy(out_s.at[b], out_hbm.at[pl.ds(chunk_base(t), C)],
                             sem_o[b])

    coords_issue(0, 0)
    coords_issue(1, 1)
    prefetch(0, 0)
    coords_issue(2, 0)

    def pipe_iter(k, carry):
        t0 = 2 * k
        t1 = 2 * k + 1
        prefetch(t1, 1)
        coords_issue(t1 + 2, 1)
        blend_store(t0, 0)
        prefetch(t0 + 2, 0)
        coords_issue(t0 + 4, 0)
        blend_store(t1, 1)
        return carry

    lax.fori_loop(0, (TPT + 1) // 2, pipe_iter, 0)

    for b in range(2):
        pltpu.make_async_copy(out_s.at[b], out_hbm.at[pl.ds(0, C)],
                              sem_o[b]).wait()


@jax.jit
def _impl(xs, ys, table):
    mesh = plsc.VectorSubcoreMesh(core_axis_name="c", subcore_axis_name="s")
    f = functools.partial(
        pl.kernel,
        mesh=mesh,
        out_type=jax.ShapeDtypeStruct((N_POINTS, FDIM), jnp.float32),
        scratch_types=[
            pltpu.VMEM((2, C), jnp.float32),
            pltpu.VMEM((2, C), jnp.float32),
            pltpu.VMEM((2, 2, C), jnp.int32),
            pltpu.VMEM((2, 4, C), jnp.float32),
            pltpu.VMEM((2, 2, C, 2 * FDIM), jnp.float32),
            pltpu.VMEM((2, C, FDIM), jnp.float32),
            pltpu.SemaphoreType.DMA,
            pltpu.SemaphoreType.DMA,
            pltpu.SemaphoreType.DMA,
            pltpu.SemaphoreType.DMA,
            pltpu.SemaphoreType.DMA,
            pltpu.SemaphoreType.DMA,
        ],
    )(_sc_body)
    return f(xs, ys, table)


def kernel(x, fm):
    xs = x[:, 0]
    ys = x[:, 1]
    t = jnp.transpose(fm[0], (1, 2, 0))
    t = jnp.pad(t, ((1, 1), (1, 1), (0, 0)))
    flat = t.reshape(WP * WP, FDIM)
    table = jnp.concatenate([flat[:-1], flat[1:]], axis=1)
    return _impl(xs, ys, table)


# speedup vs baseline: 1.3068x; 1.2676x over previous
import functools

import jax
import jax.numpy as jnp
from jax import lax
from jax.experimental import pallas as pl
from jax.experimental.pallas import tpu as pltpu
from jax.experimental.pallas import tpu_sc as plsc

FDIM = 64
GRID = 513
WP = GRID + 2
N_POINTS = 1000000
C = 128
NG = C // 16
NUM_WORKERS = 32
NCHUNK = -(-N_POINTS // C)
TPT = -(-NCHUNK // NUM_WORKERS)


def _sc_body(xs_hbm, ys_hbm, table_hbm, out_hbm,
             xs_s, ys_s, idx_s, w_s, rows_s, out_s,
             sem_c0, sem_c1, sem_g0, sem_g1, sem_o0, sem_o1):
    wid = lax.axis_index("s") * 2 + lax.axis_index("c")
    sem_c = [sem_c0, sem_c1]
    sem_g = [sem_g0, sem_g1]
    sem_o = [sem_o0, sem_o1]

    def chunk_base(t):
        return jnp.minimum((t * NUM_WORKERS + wid) * C, N_POINTS - C)

    def active(t):
        return (t * NUM_WORKERS + wid) < NCHUNK

    def coords_issue(t, b):
        @pl.when(active(t))
        def _():
            base = chunk_base(t)
            pltpu.async_copy(xs_hbm.at[pl.ds(base, C)], xs_s.at[b], sem_c[b])
            pltpu.async_copy(ys_hbm.at[pl.ds(base, C)], ys_s.at[b], sem_c[b])

    def prefetch(t, b):
        @pl.when(active(t))
        def _():
            pltpu.make_async_copy(xs_hbm.at[pl.ds(0, C)], xs_s.at[b],
                                  sem_c[b]).wait()
            pltpu.make_async_copy(ys_hbm.at[pl.ds(0, C)], ys_s.at[b],
                                  sem_c[b]).wait()
            for gi in range(NG):
                sl = pl.ds(gi * 16, 16)
                gx = xs_s[b, sl]
                gy = ys_s[b, sl]
                izx = ((gx + 1.0) * 513.0 - 1.0) * 0.5 + 1.0
                izy = ((gy + 1.0) * 513.0 - 1.0) * 0.5 + 1.0
                x0 = izx.astype(jnp.int32)
                y0 = izy.astype(jnp.int32)
                wx1 = izx - x0.astype(jnp.float32)
                wy1 = izy - y0.astype(jnp.float32)
                wx0 = 1.0 - wx1
                wy0 = 1.0 - wy1
                x0 = jnp.clip(x0, 0, WP - 2)
                y0 = jnp.clip(y0, 0, WP - 2)
                i00 = y0 * WP + x0
                idx_s[b, 0, sl] = i00
                idx_s[b, 1, sl] = i00 + WP
                w_s[b, 0, sl] = wx0 * wy0
                w_s[b, 1, sl] = wx1 * wy0
                w_s[b, 2, sl] = wx0 * wy1
                w_s[b, 3, sl] = wx1 * wy1
            for c in range(2):
                pltpu.async_copy(table_hbm.at[idx_s.at[b, c]],
                                 rows_s.at[b, c], sem_g[b])

    def blend_store(t, b):
        @pl.when(active(t))
        def _():
            for c in range(2):
                pltpu.make_async_copy(table_hbm.at[idx_s.at[b, c]],
                                      rows_s.at[b, c], sem_g[b]).wait()

            @pl.when(t >= 2)
            def _():
                pltpu.make_async_copy(out_s.at[b],
                                      out_hbm.at[pl.ds(0, C)], sem_o[b]).wait()

            def g_body(gi, _):
                p0 = gi * 16
                w0v = w_s[b, 0, pl.ds(p0, 16)]
                w1v = w_s[b, 1, pl.ds(p0, 16)]
                w2v = w_s[b, 2, pl.ds(p0, 16)]
                w3v = w_s[b, 3, pl.ds(p0, 16)]
                for lane in range(16):
                    p = p0 + lane
                    w0 = w0v[lane]
                    w1 = w1v[lane]
                    w2 = w2v[lane]
                    w3 = w3v[lane]
                    for jc in range(FDIM // 16):
                        sl = pl.ds(jc * 16, 16)
                        sl1 = pl.ds(FDIM + jc * 16, 16)
                        v0 = rows_s[b, 0, p, sl]
                        v1 = rows_s[b, 0, p, sl1]
                        v2 = rows_s[b, 1, p, sl]
                        v3 = rows_s[b, 1, p, sl1]
                        out_s[b, p, sl] = (v0 * w0 + v1 * w1
                                           + v2 * w2 + v3 * w3)
                return 0

            lax.fori_loop(0, NG, g_body, 0)
            pltpu.async_copy(out_s.at[b], out_hbm.at[pl.ds(chunk_base(t), C)],
                             sem_o[b])

    coords_issue(0, 0)
    coords_issue(1, 1)
    prefetch(0, 0)
    coords_issue(2, 0)

    def pipe_iter(k, carry):
        t0 = 2 * k
        t1 = 2 * k + 1
        prefetch(t1, 1)
        coords_issue(t1 + 2, 1)
        blend_store(t0, 0)
        prefetch(t0 + 2, 0)
        coords_issue(t0 + 4, 0)
        blend_store(t1, 1)
        return carry

    lax.fori_loop(0, (TPT + 1) // 2, pipe_iter, 0)

    for b in range(2):
        pltpu.make_async_copy(out_s.at[b], out_hbm.at[pl.ds(0, C)],
                              sem_o[b]).wait()


@jax.jit
def _impl(xs, ys, table):
    mesh = plsc.VectorSubcoreMesh(core_axis_name="c", subcore_axis_name="s")
    f = functools.partial(
        pl.kernel,
        mesh=mesh,
        out_type=jax.ShapeDtypeStruct((N_POINTS, FDIM), jnp.float32),
        scratch_types=[
            pltpu.VMEM((2, C), jnp.float32),
            pltpu.VMEM((2, C), jnp.float32),
            pltpu.VMEM((2, 2, C), jnp.int32),
            pltpu.VMEM((2, 4, C), jnp.float32),
            pltpu.VMEM((2, 2, C, 2 * FDIM), jnp.float32),
            pltpu.VMEM((2, C, FDIM), jnp.float32),
            pltpu.SemaphoreType.DMA,
            pltpu.SemaphoreType.DMA,
            pltpu.SemaphoreType.DMA,
            pltpu.SemaphoreType.DMA,
            pltpu.SemaphoreType.DMA,
            pltpu.SemaphoreType.DMA,
        ],
    )(_sc_body)
    return f(xs, ys, table)


YPAD = 520
XPAD = 640
YBLK = 8


def _table_body(fmp_ref, out_ref):
    for r in range(YBLK):
        t = jnp.transpose(fmp_ref[:, r, :], (1, 0))
        out_ref[pl.ds(r * WP, WP), :] = jnp.concatenate(
            [t[0:WP], t[1:WP + 1]], axis=1)


def _build_table(fm):
    fmp = jnp.pad(fm[0], ((0, 0), (1, YPAD - GRID - 1), (1, XPAD - GRID - 1)))
    return pl.pallas_call(
        _table_body,
        grid=(YPAD // YBLK,),
        in_specs=[pl.BlockSpec((FDIM, YBLK, XPAD), lambda k: (0, k, 0))],
        out_specs=pl.BlockSpec((YBLK * WP, 2 * FDIM), lambda k: (k, 0)),
        out_shape=jax.ShapeDtypeStruct((YPAD * WP, 2 * FDIM), jnp.float32),
    )(fmp)


def kernel(x, fm):
    xs = x[:, 0]
    ys = x[:, 1]
    return _impl(xs, ys, _build_table(fm))
